# Initial kernel scaffold; baseline (speedup 1.0000x reference)
import functools
import math

import jax
import jax.numpy as jnp
import numpy as np
from jax.experimental import pallas as pl
from jax.experimental.pallas import tpu as pltpu

_KPCA = 16
_TARGET = 1024
_BIG = jnp.float32(1e30)
_F43PI = np.float32(4.0 / 3.0 * np.pi)


def _score_kernel(rows_ref, pts_ref, gum_ref, par_ref, keys_ref, *, n):
    rows = rows_ref[0]
    pts = pts_ref[0]
    r = rows.shape[0]

    xa = pts[0:1, :]
    ya = pts[1:2, :]
    za = pts[2:3, :]
    xr = rows[:, 0:1]
    yr = rows[:, 1:2]
    zr = rows[:, 2:3]

    sq_a = xa * xa + ya * ya + za * za
    sq_r = xr * xr + yr * yr + zr * zr
    inner = xr * xa + yr * ya + zr * za
    d2 = (sq_r + sq_a) - 2.0 * inner
    mat = jnp.maximum(d2, 0.0)

    iota_n = jax.lax.broadcasted_iota(jnp.int32, (r, n), 1)
    iota_k = jax.lax.broadcasted_iota(jnp.int32, (r, _KPCA), 1)
    nbx = jnp.zeros((r, _KPCA), jnp.float32)
    nby = jnp.zeros((r, _KPCA), jnp.float32)
    nbz = jnp.zeros((r, _KPCA), jnp.float32)

    r2 = None
    for t in range(_KPCA + 1):
        m = jnp.min(mat, axis=1, keepdims=True)
        if t == _KPCA:
            r2 = m
            break
        am = jnp.min(jnp.where(mat == m, iota_n, n), axis=1, keepdims=True)
        oh = iota_n == am
        nx = jnp.sum(jnp.where(oh, xa, 0.0), axis=1, keepdims=True)
        ny = jnp.sum(jnp.where(oh, ya, 0.0), axis=1, keepdims=True)
        nz = jnp.sum(jnp.where(oh, za, 0.0), axis=1, keepdims=True)
        hit = iota_k == t
        nbx = jnp.where(hit, nx, nbx)
        nby = jnp.where(hit, ny, nby)
        nbz = jnp.where(hit, nz, nbz)
        mat = jnp.where(oh, _BIG, mat)

    inv_k = jnp.float32(1.0 / _KPCA)
    cx = jnp.sum(nbx, axis=1, keepdims=True) * inv_k
    cy = jnp.sum(nby, axis=1, keepdims=True) * inv_k
    cz = jnp.sum(nbz, axis=1, keepdims=True) * inv_k
    ccx = nbx - cx
    ccy = nby - cy
    ccz = nbz - cz
    cxx = jnp.sum(ccx * ccx, axis=1, keepdims=True) * inv_k
    cyy = jnp.sum(ccy * ccy, axis=1, keepdims=True) * inv_k
    czz = jnp.sum(ccz * ccz, axis=1, keepdims=True) * inv_k
    cxy = jnp.sum(ccx * ccy, axis=1, keepdims=True) * inv_k
    cxz = jnp.sum(ccx * ccz, axis=1, keepdims=True) * inv_k
    cyz = jnp.sum(ccy * ccz, axis=1, keepdims=True) * inv_k

    q = (cxx + cyy + czz) * jnp.float32(1.0 / 3.0)
    p1 = cxy * cxy + cxz * cxz + cyz * cyz
    dx = cxx - q
    dy = cyy - q
    dz = czz - q
    p2 = dx * dx + dy * dy + dz * dz + 2.0 * p1
    p = jnp.sqrt(p2 * jnp.float32(1.0 / 6.0))
    pinv = jnp.where(p > 1e-30, 1.0 / jnp.maximum(p, jnp.float32(1e-30)), 0.0)
    bxx = dx * pinv
    byy = dy * pinv
    bzz = dz * pinv
    bxy = cxy * pinv
    bxz = cxz * pinv
    byz = cyz * pinv
    detb = (bxx * (byy * bzz - byz * byz)
            - bxy * (bxy * bzz - byz * bxz)
            + bxz * (bxy * byz - byy * bxz))
    rr = jnp.clip(detb * 0.5, -1.0, 1.0)
    phi = jnp.arccos(rr) * jnp.float32(1.0 / 3.0)
    e1 = q + 2.0 * p * jnp.cos(phi)
    e3 = q + 2.0 * p * jnp.cos(phi + jnp.float32(2.0 * math.pi / 3.0))
    e2 = (3.0 * q - e1) - e3
    a1 = jnp.abs(e1)
    a2 = jnp.abs(e2)
    a3 = jnp.abs(e3)
    hi = jnp.maximum(a1, a2)
    lo = jnp.minimum(a1, a2)
    l1 = jnp.maximum(hi, a3) + 1e-8
    l2 = jnp.maximum(lo, jnp.minimum(hi, a3)) + 1e-8
    l3 = jnp.minimum(lo, a3) + 1e-8
    term1 = (l2 - l3) / l1
    term2 = (l1 - l2) / l1
    term3 = l3 / l1
    w0 = par_ref[0, 3]
    w1 = par_ref[0, 4]
    w2 = par_ref[0, 5]
    complexity = w0 * term1 + w1 * term2 + w2 * term3

    r16 = jnp.sqrt(r2)
    volume = _F43PI * ((r16 * r16) * r16 + 1e-8)
    density = jnp.float32(_KPCA) / volume
    log_density = jnp.log(density + 1e-8)
    alpha = par_ref[0, 0]
    beta = par_ref[0, 1]
    gamma = par_ref[0, 2]
    logits = (alpha * log_density + beta * complexity) + gamma
    prob = jax.nn.sigmoid(logits)
    keys = jnp.log(prob + 1e-12) + gum_ref[0]
    keys_ref[0] = keys


def _rol(a, j, n):
    return jnp.concatenate([a[:, j:], a[:, :j]], axis=1)


def _ror(a, j, n):
    return jnp.concatenate([a[:, n - j:], a[:, :n - j]], axis=1)


def _topk_kernel(keys_ref, pts_ref, idx_ref, pts_out_ref, *, n):
    keys = keys_ref[0]
    iota = jax.lax.broadcasted_iota(jnp.int32, (1, n), 1)
    idx = iota

    kk = 2
    while kk <= n:
        j = kk // 2
        while j >= 1:
            low = (iota & j) == 0
            desc = (iota & kk) == 0
            take_max = low == desc
            pk = jnp.where(low, _rol(keys, j, n), _ror(keys, j, n))
            pi = jnp.where(low, _rol(idx, j, n), _ror(idx, j, n))
            gt = (pk > keys) | ((pk == keys) & (pi < idx))
            sel = take_max == gt
            keys = jnp.where(sel, pk, keys)
            idx = jnp.where(sel, pi, idx)
            j //= 2
        kk *= 2

    ids = idx[:, :_TARGET]
    idx_ref[0] = ids

    sub = jax.lax.broadcasted_iota(jnp.int32, (n, _TARGET), 0)
    onehot = (sub == ids).astype(jnp.float32)
    pts = pts_ref[0]
    gathered = jax.lax.dot_general(
        pts, onehot, (((1,), (0,)), ((), ())),
        precision=jax.lax.Precision.HIGHEST,
        preferred_element_type=jnp.float32)
    pts_out_ref[0] = gathered


def kernel(points, alpha, beta, gamma, complexity_weights):
    b, n, c = points.shape
    rtile = 256
    pts_t = jnp.swapaxes(points, 1, 2)
    gkey = jax.random.fold_in(jax.random.key(42), 1)
    gumbel = jax.random.gumbel(gkey, (b, n), dtype=jnp.float32)
    gum3 = gumbel[..., None]
    params = jnp.concatenate([
        jnp.stack([alpha, beta, gamma]).astype(jnp.float32),
        complexity_weights.astype(jnp.float32),
        jnp.zeros((2,), jnp.float32)]).reshape(1, 8)

    keys = pl.pallas_call(
        functools.partial(_score_kernel, n=n),
        grid=(b, n // rtile),
        in_specs=[
            pl.BlockSpec((1, rtile, c), lambda bb, i: (bb, i, 0)),
            pl.BlockSpec((1, c, n), lambda bb, i: (bb, 0, 0)),
            pl.BlockSpec((1, rtile, 1), lambda bb, i: (bb, i, 0)),
            pl.BlockSpec(memory_space=pltpu.SMEM),
        ],
        out_specs=pl.BlockSpec((1, rtile, 1), lambda bb, i: (bb, i, 0)),
        out_shape=jax.ShapeDtypeStruct((b, n, 1), jnp.float32),
    )(points, pts_t, gum3, params)

    idx3, pts3 = pl.pallas_call(
        functools.partial(_topk_kernel, n=n),
        grid=(b,),
        in_specs=[
            pl.BlockSpec((1, 1, n), lambda bb: (bb, 0, 0)),
            pl.BlockSpec((1, c, n), lambda bb: (bb, 0, 0)),
        ],
        out_specs=[
            pl.BlockSpec((1, 1, _TARGET), lambda bb: (bb, 0, 0)),
            pl.BlockSpec((1, c, _TARGET), lambda bb: (bb, 0, 0)),
        ],
        out_shape=[
            jax.ShapeDtypeStruct((b, 1, _TARGET), jnp.int32),
            jax.ShapeDtypeStruct((b, c, _TARGET), jnp.float32),
        ],
    )(keys.reshape(b, 1, n), pts_t)

    sampled_indices = idx3[:, 0, :]
    sampled_points = jnp.swapaxes(pts3, 1, 2)
    return sampled_points, sampled_indices


# speedup vs baseline: 20.5146x; 20.5146x over previous
import functools
import math

import jax
import jax.numpy as jnp
import numpy as np
from jax.experimental import pallas as pl
from jax.experimental.pallas import tpu as pltpu

_KPCA = 16
_TARGET = 1024
_BIG = 1e30
_F43PI = np.float32(4.0 / 3.0 * np.pi)


def _jrot(app, aqq, apq):
    nz = jnp.abs(apq) > 0.0
    theta = (aqq - app) / jnp.where(nz, 2.0 * apq, 1.0)
    sgn = jnp.where(theta >= 0.0, 1.0, -1.0)
    t = sgn / (jnp.abs(theta) + jnp.sqrt(theta * theta + 1.0))
    t = jnp.where(nz, t, 0.0)
    c = 1.0 / jnp.sqrt(t * t + 1.0)
    s = t * c
    return c, s, t


def _score_kernel(rows_ref, pts_ref, par_ref, r2_ref, cx_ref, *, n):
    rows = rows_ref[0]
    pts = pts_ref[0]
    r = rows.shape[0]

    xa = pts[0:1, :]
    ya = pts[1:2, :]
    za = pts[2:3, :]
    xr = rows[:, 0:1]
    yr = rows[:, 1:2]
    zr = rows[:, 2:3]

    xab = xa.astype(jnp.bfloat16).astype(jnp.float32)
    yab = ya.astype(jnp.bfloat16).astype(jnp.float32)
    zab = za.astype(jnp.bfloat16).astype(jnp.float32)
    xrb = xr.astype(jnp.bfloat16).astype(jnp.float32)
    yrb = yr.astype(jnp.bfloat16).astype(jnp.float32)
    zrb = zr.astype(jnp.bfloat16).astype(jnp.float32)
    sq_a = xa * xa + ya * ya + za * za
    sq_r = xr * xr + yr * yr + zr * zr
    inner = xrb * xab + yrb * yab + zrb * zab
    d2 = (sq_r + sq_a) - 2.0 * inner
    mat = jnp.maximum(d2, 0.0)

    iota_n = jax.lax.broadcasted_iota(jnp.int32, (r, n), 1)
    iota_k = jax.lax.broadcasted_iota(jnp.int32, (r, _KPCA), 1)
    nbx = jnp.zeros((r, _KPCA), jnp.float32)
    nby = jnp.zeros((r, _KPCA), jnp.float32)
    nbz = jnp.zeros((r, _KPCA), jnp.float32)

    r2 = None
    for t in range(_KPCA + 1):
        m = jnp.min(mat, axis=1, keepdims=True)
        if t == _KPCA:
            r2 = m
            break
        am = jnp.min(jnp.where(mat == m, iota_n, n), axis=1, keepdims=True)
        oh = iota_n == am
        nx = jnp.sum(jnp.where(oh, xa, 0.0), axis=1, keepdims=True)
        ny = jnp.sum(jnp.where(oh, ya, 0.0), axis=1, keepdims=True)
        nz = jnp.sum(jnp.where(oh, za, 0.0), axis=1, keepdims=True)
        hit = iota_k == t
        nbx = jnp.where(hit, nx, nbx)
        nby = jnp.where(hit, ny, nby)
        nbz = jnp.where(hit, nz, nbz)
        mat = jnp.where(oh, _BIG, mat)

    inv_k = np.float32(1.0 / _KPCA)
    cx = jnp.sum(nbx, axis=1, keepdims=True) * inv_k
    cy = jnp.sum(nby, axis=1, keepdims=True) * inv_k
    cz = jnp.sum(nbz, axis=1, keepdims=True) * inv_k
    ccx = nbx - cx
    ccy = nby - cy
    ccz = nbz - cz
    cxx = jnp.sum(ccx * ccx, axis=1, keepdims=True) * inv_k
    cyy = jnp.sum(ccy * ccy, axis=1, keepdims=True) * inv_k
    czz = jnp.sum(ccz * ccz, axis=1, keepdims=True) * inv_k
    cxy = jnp.sum(ccx * ccy, axis=1, keepdims=True) * inv_k
    cxz = jnp.sum(ccx * ccz, axis=1, keepdims=True) * inv_k
    cyz = jnp.sum(ccy * ccz, axis=1, keepdims=True) * inv_k

    a00, a11, a22 = cxx, cyy, czz
    a01, a02, a12 = cxy, cxz, cyz
    for _ in range(5):
        c, s, t = _jrot(a00, a11, a01)
        a00 = a00 - t * a01
        a11 = a11 + t * a01
        a02, a12 = c * a02 - s * a12, s * a02 + c * a12
        a01 = jnp.zeros_like(a01)
        c, s, t = _jrot(a00, a22, a02)
        a00 = a00 - t * a02
        a22 = a22 + t * a02
        a01, a12 = c * a01 - s * a12, s * a01 + c * a12
        a02 = jnp.zeros_like(a02)
        c, s, t = _jrot(a11, a22, a12)
        a11 = a11 - t * a12
        a22 = a22 + t * a12
        a01, a02 = c * a01 - s * a02, s * a01 + c * a02
        a12 = jnp.zeros_like(a12)
    a1 = jnp.abs(a00)
    a2 = jnp.abs(a11)
    a3 = jnp.abs(a22)
    hi = jnp.maximum(a1, a2)
    lo = jnp.minimum(a1, a2)
    l1 = jnp.maximum(hi, a3) + 1e-8
    l2 = jnp.maximum(lo, jnp.minimum(hi, a3)) + 1e-8
    l3 = jnp.minimum(lo, a3) + 1e-8
    term1 = (l2 - l3) / l1
    term2 = (l1 - l2) / l1
    term3 = l3 / l1
    w0 = par_ref[0, 0]
    w1 = par_ref[0, 1]
    w2 = par_ref[0, 2]
    complexity = w0 * term1 + w1 * term2 + w2 * term3

    r2_ref[0] = r2
    cx_ref[0] = complexity


def _rol(a, j, n):
    return jnp.concatenate([a[:, j:], a[:, :j]], axis=1)


def _ror(a, j, n):
    return jnp.concatenate([a[:, n - j:], a[:, :n - j]], axis=1)


def _topk_kernel(keys_ref, pts_ref, idx_ref, pts_out_ref, *, n):
    keys = keys_ref[0]
    iota = jax.lax.broadcasted_iota(jnp.int32, (1, n), 1)
    idx = iota

    kk = 2
    while kk <= n:
        j = kk // 2
        while j >= 1:
            low = (iota & j) == 0
            desc = (iota & kk) == 0
            take_max = low == desc
            pk = jnp.where(low, _rol(keys, j, n), _ror(keys, j, n))
            pi = jnp.where(low, _rol(idx, j, n), _ror(idx, j, n))
            gt = (pk > keys) | ((pk == keys) & (pi < idx))
            sel = take_max == gt
            keys = jnp.where(sel, pk, keys)
            idx = jnp.where(sel, pi, idx)
            j //= 2
        kk *= 2

    ids = idx[:, :_TARGET]
    idx_ref[0] = ids

    sub = jax.lax.broadcasted_iota(jnp.int32, (n, _TARGET), 0)
    onehot = (sub == ids).astype(jnp.float32)
    pts = pts_ref[0]
    gathered = jax.lax.dot_general(
        pts, onehot, (((1,), (0,)), ((), ())),
        precision=jax.lax.Precision.HIGHEST,
        preferred_element_type=jnp.float32)
    pts_out_ref[0] = gathered


def kernel(points, alpha, beta, gamma, complexity_weights):
    b, n, c = points.shape
    rtile = 256
    pts_t = jnp.swapaxes(points, 1, 2)
    params = jnp.concatenate([
        complexity_weights.astype(jnp.float32),
        jnp.zeros((5,), jnp.float32)]).reshape(1, 8)

    r2, cx = pl.pallas_call(
        functools.partial(_score_kernel, n=n),
        grid=(b, n // rtile),
        in_specs=[
            pl.BlockSpec((1, rtile, c), lambda bb, i: (bb, i, 0)),
            pl.BlockSpec((1, c, n), lambda bb, i: (bb, 0, 0)),
            pl.BlockSpec(memory_space=pltpu.SMEM),
        ],
        out_specs=[pl.BlockSpec((1, rtile, 1), lambda bb, i: (bb, i, 0))] * 2,
        out_shape=[jax.ShapeDtypeStruct((b, n, 1), jnp.float32)] * 2,
    )(points, pts_t, params)

    r_k = jnp.sqrt(r2[..., 0])
    volume = 4.0 / 3.0 * np.pi * (r_k ** 3 + 1e-08)
    density_16 = 16 / volume
    log_density = jnp.log(density_16 + 1e-08)
    logits = alpha * log_density + beta * cx[..., 0] + gamma
    probabilities = jax.nn.sigmoid(logits)
    gkey = jax.random.fold_in(jax.random.key(42), 1)
    gumbel = jax.random.gumbel(gkey, probabilities.shape,
                               dtype=probabilities.dtype)
    keys = jnp.log(probabilities + 1e-12) + gumbel

    idx3, pts3 = pl.pallas_call(
        functools.partial(_topk_kernel, n=n),
        grid=(b,),
        in_specs=[
            pl.BlockSpec((1, 1, n), lambda bb: (bb, 0, 0)),
            pl.BlockSpec((1, c, n), lambda bb: (bb, 0, 0)),
        ],
        out_specs=[
            pl.BlockSpec((1, 1, _TARGET), lambda bb: (bb, 0, 0)),
            pl.BlockSpec((1, c, _TARGET), lambda bb: (bb, 0, 0)),
        ],
        out_shape=[
            jax.ShapeDtypeStruct((b, 1, _TARGET), jnp.int32),
            jax.ShapeDtypeStruct((b, c, _TARGET), jnp.float32),
        ],
    )(keys.reshape(b, 1, n), pts_t)

    sampled_indices = idx3[:, 0, :]
    sampled_points = jnp.swapaxes(pts3, 1, 2)
    return sampled_points, sampled_indices
